# baseline (speedup 1.0000x reference)
import jax
import jax.numpy as jnp
from jax.experimental import pallas as pl
from jax.experimental.pallas import tpu as pltpu

TOKENS = 32768
D_MODEL = 1024
NUM_EXPERTS = 64
THRESHOLD = 0.5
TILE = 2048
NT = TOKENS // TILE
HALF = TILE // 2


def _gate_kernel(x_ref, w_ref, b_ref, gp_ref):
    logits = jax.lax.dot_general(
        x_ref[...], w_ref[...],
        dimension_numbers=(((1,), (1,)), ((), ())),
        preferred_element_type=jnp.float32,
    )
    logits = logits + b_ref[...]
    probs = jax.nn.sigmoid(logits)
    mask = (probs > THRESHOLD).astype(jnp.float32)
    gated = probs * mask
    gp_ref[...] = jnp.concatenate([gated[:HALF], gated[HALF:]], axis=1)


def kernel(x, W, b):
    b2 = b.reshape(1, NUM_EXPERTS)
    gp = pl.pallas_call(
        _gate_kernel,
        grid=(NT,),
        in_specs=[
            pl.BlockSpec((TILE, D_MODEL), lambda i: (i, 0)),
            pl.BlockSpec((NUM_EXPERTS, D_MODEL), lambda i: (0, 0)),
            pl.BlockSpec((1, NUM_EXPERTS), lambda i: (0, 0)),
        ],
        out_specs=pl.BlockSpec((HALF, 2 * NUM_EXPERTS), lambda i: (i, 0)),
        out_shape=jax.ShapeDtypeStruct((TOKENS // 2, 2 * NUM_EXPERTS), jnp.float32),
        compiler_params=pltpu.CompilerParams(
            dimension_semantics=("arbitrary",),
        ),
    )(x, W, b2)
    g3 = gp.reshape(NT, HALF, 2 * NUM_EXPERTS)
    gated = jnp.concatenate(
        [g3[:, :, :NUM_EXPERTS], g3[:, :, NUM_EXPERTS:]], axis=1
    ).reshape(TOKENS, NUM_EXPERTS)
    mask = (gated > 0.0).astype(jnp.float32)
    return gated, mask
